# Initial kernel scaffold; baseline (speedup 1.0000x reference)
import functools

import jax
import jax.numpy as jnp
from jax import lax
from jax.experimental import pallas as pl
from jax.experimental.pallas import tpu as pltpu
from jax.experimental.pallas import tpu_sc as plsc

_NUM_CODES = 100000
_NUM_CLUSTERS = 1000
_NUM_PARENTS = 50
_SUB = 32
_BATCH, _HIST = 4096, 200
_N = _BATCH * _HIST
_C = 128
_NCHUNKS = _N // _C


@functools.lru_cache(maxsize=None)
def _build():
    info = plsc.get_sparse_core_info()
    nc, ns = info.num_cores, info.num_subcores
    nw = nc * ns
    chunks_w = _NCHUNKS // nw
    nbuf = 4
    kbatch = 8

    mesh = plsc.VectorSubcoreMesh(core_axis_name="c", subcore_axis_name="s")

    @functools.partial(
        pl.kernel,
        out_type=jax.ShapeDtypeStruct((_N, 3 * _SUB), jnp.float32),
        mesh=mesh,
        scratch_types=[
            pltpu.VMEM((chunks_w, _C), jnp.int32),
            pltpu.VMEM((chunks_w, _C), jnp.int32),
            pltpu.VMEM((chunks_w, _C), jnp.int32),
            pltpu.VMEM((nbuf, _C, _SUB), jnp.float32),
            pltpu.VMEM((nbuf, _C, _SUB), jnp.float32),
            pltpu.VMEM((nbuf, _C, _SUB), jnp.float32),
            pltpu.SemaphoreType.DMA,
            pltpu.SemaphoreType.DMA,
            pltpu.SemaphoreType.DMA,
        ],
    )
    def enc(codes2_hbm, c2c_hbm, c2p_hbm, cemb_hbm, clemb_hbm, pemb_hbm,
            out_hbm, codes_v, cid_v, pid_v, crow_v, lrow_v, prow_v,
            gsem, rsem, wsem):
        wid = lax.axis_index("s") * nc + lax.axis_index("c")
        g0 = wid * chunks_w

        pltpu.sync_copy(codes2_hbm.at[pl.ds(g0, chunks_w), :], codes_v)

        def id_batch(t, carry):
            ds = []
            for b in range(kbatch):
                g = t * kbatch + b
                idx = codes_v.at[g]
                ds.append(pltpu.async_copy(c2c_hbm.at[idx], cid_v.at[g], gsem))
                ds.append(pltpu.async_copy(c2p_hbm.at[idx], pid_v.at[g], gsem))
            for d in ds:
                d.wait()
            return carry
        lax.fori_loop(0, chunks_w // kbatch, id_batch, 0)

        def row_batch(t, carry):
            gds = []
            for b in range(nbuf):
                g = t * nbuf + b
                gds.append(pltpu.async_copy(
                    cemb_hbm.at[codes_v.at[g]], crow_v.at[b], rsem))
                gds.append(pltpu.async_copy(
                    clemb_hbm.at[cid_v.at[g]], lrow_v.at[b], rsem))
                gds.append(pltpu.async_copy(
                    pemb_hbm.at[pid_v.at[g]], prow_v.at[b], rsem))
            wds = []
            for b in range(nbuf):
                g = t * nbuf + b
                base = (g0 + g) * _C
                gds[3 * b].wait()
                wds.append(pltpu.async_copy(
                    crow_v.at[b], out_hbm.at[pl.ds(base, _C), pl.ds(0, _SUB)],
                    wsem))
                gds[3 * b + 1].wait()
                wds.append(pltpu.async_copy(
                    lrow_v.at[b],
                    out_hbm.at[pl.ds(base, _C), pl.ds(_SUB, _SUB)], wsem))
                gds[3 * b + 2].wait()
                wds.append(pltpu.async_copy(
                    prow_v.at[b],
                    out_hbm.at[pl.ds(base, _C), pl.ds(2 * _SUB, _SUB)], wsem))
            for d in wds:
                d.wait()
            return carry
        lax.fori_loop(0, chunks_w // nbuf, row_batch, 0)

    return enc


def kernel(codes, code_to_cluster, code_to_parent, code_emb, cluster_emb,
           parent_emb):
    codes2 = codes.reshape(_NCHUNKS, _C)
    out = _build()(codes2, code_to_cluster, code_to_parent, code_emb,
                   cluster_emb, parent_emb)
    return out.reshape(_BATCH, _HIST, 3 * _SUB)


# speedup vs baseline: 13.6432x; 13.6432x over previous
import functools

import jax
import jax.numpy as jnp
from jax import lax
from jax.experimental import pallas as pl
from jax.experimental.pallas import tpu as pltpu
from jax.experimental.pallas import tpu_sc as plsc

_NUM_CODES = 100000
_NUM_CLUSTERS = 1000
_NUM_PARENTS = 50
_SUB = 32
_BATCH, _HIST = 4096, 200
_N = _BATCH * _HIST
_C = 128
_NCHUNKS = _N // _C


@functools.lru_cache(maxsize=None)
def _build():
    info = plsc.get_sparse_core_info()
    nc, ns = info.num_cores, info.num_subcores
    nw = nc * ns
    chunks_w = _NCHUNKS // nw
    nbuf = 4
    kbatch = 8

    mesh = plsc.VectorSubcoreMesh(core_axis_name="c", subcore_axis_name="s")

    @functools.partial(
        pl.kernel,
        out_type=jax.ShapeDtypeStruct((_N, 3 * _SUB), jnp.float32),
        mesh=mesh,
        compiler_params=pltpu.CompilerParams(use_tc_tiling_on_sc=False),
        scratch_types=[
            pltpu.VMEM((chunks_w, _C), jnp.int32),
            pltpu.VMEM((chunks_w, _C), jnp.int32),
            pltpu.VMEM((chunks_w, _C), jnp.int32),
            pltpu.VMEM((nbuf, _C, _SUB), jnp.float32),
            pltpu.VMEM((nbuf, _C, _SUB), jnp.float32),
            pltpu.VMEM((nbuf, _C, _SUB), jnp.float32),
            pltpu.SemaphoreType.DMA,
            pltpu.SemaphoreType.DMA,
            pltpu.SemaphoreType.DMA,
        ],
    )
    def enc(codes2_hbm, c2c_hbm, c2p_hbm, cemb_hbm, clemb_hbm, pemb_hbm,
            out_hbm, codes_v, cid_v, pid_v, crow_v, lrow_v, prow_v,
            gsem, rsem, wsem):
        wid = lax.axis_index("s") * nc + lax.axis_index("c")
        g0 = wid * chunks_w

        pltpu.sync_copy(codes2_hbm.at[pl.ds(g0, chunks_w), :], codes_v)

        def id_batch(t, carry):
            ds = []
            for b in range(kbatch):
                g = t * kbatch + b
                idx = codes_v.at[g]
                ds.append(pltpu.async_copy(c2c_hbm.at[idx], cid_v.at[g], gsem))
                ds.append(pltpu.async_copy(c2p_hbm.at[idx], pid_v.at[g], gsem))
            for d in ds:
                d.wait()
            return carry
        lax.fori_loop(0, chunks_w // kbatch, id_batch, 0)

        def row_batch(t, carry):
            gds = []
            for b in range(nbuf):
                g = t * nbuf + b
                gds.append(pltpu.async_copy(
                    cemb_hbm.at[codes_v.at[g]], crow_v.at[b], rsem))
                gds.append(pltpu.async_copy(
                    clemb_hbm.at[cid_v.at[g]], lrow_v.at[b], rsem))
                gds.append(pltpu.async_copy(
                    pemb_hbm.at[pid_v.at[g]], prow_v.at[b], rsem))
            wds = []
            for b in range(nbuf):
                g = t * nbuf + b
                base = (g0 + g) * _C
                gds[3 * b].wait()
                wds.append(pltpu.async_copy(
                    crow_v.at[b], out_hbm.at[pl.ds(base, _C), pl.ds(0, _SUB)],
                    wsem))
                gds[3 * b + 1].wait()
                wds.append(pltpu.async_copy(
                    lrow_v.at[b],
                    out_hbm.at[pl.ds(base, _C), pl.ds(_SUB, _SUB)], wsem))
                gds[3 * b + 2].wait()
                wds.append(pltpu.async_copy(
                    prow_v.at[b],
                    out_hbm.at[pl.ds(base, _C), pl.ds(2 * _SUB, _SUB)], wsem))
            for d in wds:
                d.wait()
            return carry
        lax.fori_loop(0, chunks_w // nbuf, row_batch, 0)

    return enc


def kernel(codes, code_to_cluster, code_to_parent, code_emb, cluster_emb,
           parent_emb):
    codes2 = codes.reshape(_NCHUNKS, _C)
    out = _build()(codes2, code_to_cluster, code_to_parent, code_emb,
                   cluster_emb, parent_emb)
    return out.reshape(_BATCH, _HIST, 3 * _SUB)


# speedup vs baseline: 13.6702x; 1.0020x over previous
import functools

import jax
import jax.numpy as jnp
from jax import lax
from jax.experimental import pallas as pl
from jax.experimental.pallas import tpu as pltpu
from jax.experimental.pallas import tpu_sc as plsc

_NUM_CODES = 100000
_NUM_CLUSTERS = 1000
_NUM_PARENTS = 50
_SUB = 32
_BATCH, _HIST = 4096, 200
_N = _BATCH * _HIST
_C = 128
_NCHUNKS = _N // _C


@functools.lru_cache(maxsize=None)
def _build():
    info = plsc.get_sparse_core_info()
    nc, ns = info.num_cores, info.num_subcores
    nw = nc * ns
    chunks_w = _NCHUNKS // nw
    nbuf = 4
    kbatch = 8

    mesh = plsc.VectorSubcoreMesh(core_axis_name="c", subcore_axis_name="s")

    @functools.partial(
        pl.kernel,
        out_type=jax.ShapeDtypeStruct((_N, 3 * _SUB), jnp.float32),
        mesh=mesh,
        compiler_params=pltpu.CompilerParams(use_tc_tiling_on_sc=False),
        scratch_types=[
            pltpu.VMEM((chunks_w, _C), jnp.int32),
            pltpu.VMEM((chunks_w, _C), jnp.int32),
            pltpu.VMEM((chunks_w, _C), jnp.int32),
            pltpu.VMEM((nbuf, _C, _SUB), jnp.float32),
            pltpu.VMEM((nbuf, _C, _SUB), jnp.float32),
            pltpu.VMEM((nbuf, _C, _SUB), jnp.float32),
            pltpu.SemaphoreType.DMA,
            [pltpu.SemaphoreType.DMA] * nbuf,
            [pltpu.SemaphoreType.DMA] * nbuf,
        ],
    )
    def enc(codes2_hbm, c2c_hbm, c2p_hbm, cemb_hbm, clemb_hbm, pemb_hbm,
            out_hbm, codes_v, cid_v, pid_v, crow_v, lrow_v, prow_v,
            gsem, rsems, wsems):
        wid = lax.axis_index("s") * nc + lax.axis_index("c")
        g0 = wid * chunks_w

        pltpu.sync_copy(codes2_hbm.at[pl.ds(g0, chunks_w), :], codes_v)

        def id_drain():
            for _ in range(2 * kbatch):
                pltpu.make_async_copy(
                    c2c_hbm.at[codes_v.at[0]], cid_v.at[0], gsem).wait()

        def id_batch(t, carry):
            for b in range(kbatch):
                g = t * kbatch + b
                idx = codes_v.at[g]
                pltpu.async_copy(c2c_hbm.at[idx], cid_v.at[g], gsem)
                pltpu.async_copy(c2p_hbm.at[idx], pid_v.at[g], gsem)
            @pl.when(t != 0)
            def _():
                id_drain()
            return carry
        lax.fori_loop(0, chunks_w // kbatch, id_batch, 0)
        id_drain()

        def out_band(base, k):
            return out_hbm.at[pl.ds(base, _C), pl.ds(k * _SUB, _SUB)]

        def wait_writes(b):
            pltpu.make_async_copy(crow_v.at[b], out_band(0, 0), wsems[b]).wait()
            pltpu.make_async_copy(lrow_v.at[b], out_band(0, 1), wsems[b]).wait()
            pltpu.make_async_copy(prow_v.at[b], out_band(0, 2), wsems[b]).wait()

        def row_batch(t, carry):
            gds = []
            for b in range(nbuf):
                g = t * nbuf + b
                @pl.when(t != 0)
                def _(b=b):
                    wait_writes(b)
                gds.append(pltpu.async_copy(
                    cemb_hbm.at[codes_v.at[g]], crow_v.at[b], rsems[b]))
                gds.append(pltpu.async_copy(
                    clemb_hbm.at[cid_v.at[g]], lrow_v.at[b], rsems[b]))
                gds.append(pltpu.async_copy(
                    pemb_hbm.at[pid_v.at[g]], prow_v.at[b], rsems[b]))
            for b in range(nbuf):
                g = t * nbuf + b
                base = (g0 + g) * _C
                gds[3 * b].wait()
                pltpu.async_copy(crow_v.at[b], out_band(base, 0), wsems[b])
                gds[3 * b + 1].wait()
                pltpu.async_copy(lrow_v.at[b], out_band(base, 1), wsems[b])
                gds[3 * b + 2].wait()
                pltpu.async_copy(prow_v.at[b], out_band(base, 2), wsems[b])
            return carry
        lax.fori_loop(0, chunks_w // nbuf, row_batch, 0)
        for b in range(nbuf):
            wait_writes(b)

    return enc


def kernel(codes, code_to_cluster, code_to_parent, code_emb, cluster_emb,
           parent_emb):
    codes2 = codes.reshape(_NCHUNKS, _C)
    out = _build()(codes2, code_to_cluster, code_to_parent, code_emb,
                   cluster_emb, parent_emb)
    return out.reshape(_BATCH, _HIST, 3 * _SUB)


# speedup vs baseline: 13.6755x; 1.0004x over previous
import functools

import jax
import jax.numpy as jnp
from jax import lax
from jax.experimental import pallas as pl
from jax.experimental.pallas import tpu as pltpu
from jax.experimental.pallas import tpu_sc as plsc

_NUM_CODES = 100000
_NUM_CLUSTERS = 1000
_NUM_PARENTS = 50
_SUB = 32
_BATCH, _HIST = 4096, 200
_N = _BATCH * _HIST
_C = 128
_NCHUNKS = _N // _C


@functools.lru_cache(maxsize=None)
def _build():
    info = plsc.get_sparse_core_info()
    nc, ns = info.num_cores, info.num_subcores
    nw = nc * ns
    chunks_w = _NCHUNKS // nw
    nbuf = 4
    kbatch = 8

    mesh = plsc.VectorSubcoreMesh(core_axis_name="c", subcore_axis_name="s")

    @functools.partial(
        pl.kernel,
        out_type=jax.ShapeDtypeStruct((_N, 3 * _SUB), jnp.float32),
        mesh=mesh,
        compiler_params=pltpu.CompilerParams(use_tc_tiling_on_sc=False),
        scratch_types=[
            pltpu.VMEM((chunks_w, _C), jnp.int32),
            pltpu.VMEM((chunks_w, _C), jnp.int32),
            pltpu.VMEM((chunks_w, _C), jnp.int32),
            pltpu.VMEM((nbuf, _C, _SUB), jnp.float32),
            pltpu.VMEM((nbuf, _C, _SUB), jnp.float32),
            pltpu.VMEM((nbuf, _C, _SUB), jnp.float32),
            pltpu.SemaphoreType.DMA,
            [pltpu.SemaphoreType.DMA] * nbuf,
            [pltpu.SemaphoreType.DMA] * nbuf,
        ],
    )
    def enc(codes2_hbm, c2c_hbm, c2p_hbm, cemb_hbm, clemb_hbm, pemb_hbm,
            out_hbm, codes_v, cid_v, pid_v, crow_v, lrow_v, prow_v,
            gsem, rsems, wsems):
        wid = lax.axis_index("s") * nc + lax.axis_index("c")
        g0 = wid * chunks_w

        with jax.named_scope("p1_codes"):
            pltpu.sync_copy(codes2_hbm.at[pl.ds(g0, chunks_w), :], codes_v)

        def id_drain():
            for _ in range(2 * kbatch):
                pltpu.make_async_copy(
                    c2c_hbm.at[codes_v.at[0]], cid_v.at[0], gsem).wait()

        def id_batch(t, carry):
            for b in range(kbatch):
                g = t * kbatch + b
                idx = codes_v.at[g]
                pltpu.async_copy(c2c_hbm.at[idx], cid_v.at[g], gsem)
                pltpu.async_copy(c2p_hbm.at[idx], pid_v.at[g], gsem)
            @pl.when(t != 0)
            def _():
                id_drain()
            return carry
        with jax.named_scope("p2_ids"):
            lax.fori_loop(0, chunks_w // kbatch, id_batch, 0)
            id_drain()

        def out_band(base, k):
            return out_hbm.at[pl.ds(base, _C), pl.ds(k * _SUB, _SUB)]

        def wait_writes(b):
            pltpu.make_async_copy(crow_v.at[b], out_band(0, 0), wsems[b]).wait()
            pltpu.make_async_copy(lrow_v.at[b], out_band(0, 1), wsems[b]).wait()
            pltpu.make_async_copy(prow_v.at[b], out_band(0, 2), wsems[b]).wait()

        def row_batch(t, carry):
            gds = []
            for b in range(nbuf):
                g = t * nbuf + b
                @pl.when(t != 0)
                def _(b=b):
                    wait_writes(b)
                gds.append(pltpu.async_copy(
                    cemb_hbm.at[codes_v.at[g]], crow_v.at[b], rsems[b]))
                gds.append(pltpu.async_copy(
                    clemb_hbm.at[cid_v.at[g]], lrow_v.at[b], rsems[b]))
                gds.append(pltpu.async_copy(
                    pemb_hbm.at[pid_v.at[g]], prow_v.at[b], rsems[b]))
            for b in range(nbuf):
                g = t * nbuf + b
                base = (g0 + g) * _C
                gds[3 * b].wait()
                pltpu.async_copy(crow_v.at[b], out_band(base, 0), wsems[b])
                gds[3 * b + 1].wait()
                pltpu.async_copy(lrow_v.at[b], out_band(base, 1), wsems[b])
                gds[3 * b + 2].wait()
                pltpu.async_copy(prow_v.at[b], out_band(base, 2), wsems[b])
            return carry
        with jax.named_scope("p3_rows"):
            lax.fori_loop(0, chunks_w // nbuf, row_batch, 0)
            for b in range(nbuf):
                wait_writes(b)

    return enc


def kernel(codes, code_to_cluster, code_to_parent, code_emb, cluster_emb,
           parent_emb):
    codes2 = codes.reshape(_NCHUNKS, _C)
    out = _build()(codes2, code_to_cluster, code_to_parent, code_emb,
                   cluster_emb, parent_emb)
    return out.reshape(_BATCH, _HIST, 3 * _SUB)
